# baseline (speedup 1.0000x reference)
import functools

import jax
import jax.numpy as jnp
from jax import lax
from jax.experimental import pallas as pl
from jax.experimental.pallas import tpu as pltpu
from jax.experimental.pallas import tpu_sc as plsc

MAXLEN = 200
D = 64
L = 16
DG = D // L
BBLK = 128
RUN = 4

_info = plsc.get_sparse_core_info()
NC = _info.num_cores
NS = _info.num_subcores
NW = NC * NS


def _make_kernel(vocab, batch):
    assert batch == NW * BBLK

    mesh = plsc.VectorSubcoreMesh(core_axis_name="c", subcore_axis_name="s")

    @functools.partial(
        pl.kernel,
        mesh=mesh,
        compiler_params=pltpu.CompilerParams(
            use_tc_tiling_on_sc=False, needs_layout_passes=False
        ),
        out_type=jax.ShapeDtypeStruct((MAXLEN, D // 8, NW, 8 * BBLK), jnp.float32),
        scratch_types=[
            pltpu.VMEM((MAXLEN, D), jnp.float32),
            pltpu.VMEM((MAXLEN, BBLK), jnp.int32),
            pltpu.VMEM((BBLK, 2 * D), jnp.float32),
            pltpu.VMEM((BBLK, 2 * D), jnp.float32),
            pltpu.VMEM((D // 8, 8 * BBLK + 8), jnp.float32),
            pltpu.VMEM((D // 8, 8 * BBLK + 8), jnp.float32),
            pltpu.SemaphoreType.DMA,
            pltpu.SemaphoreType.DMA,
            pltpu.SemaphoreType.DMA,
            pltpu.SemaphoreType.DMA,
        ],
    )
    def tok_pos_kernel(xt_hbm, tab_hbm, pos_hbm, out_hbm,
                       pos_v, idx_v, dst0, dst1, out0, out1,
                       g0, g1, w0, w1):
        wid = lax.axis_index("s") * NC + lax.axis_index("c")
        b0 = wid * BBLK
        dsts = (dst0, dst1)
        outs = (out0, out1)
        gsems = (g0, g1)
        wsems = (w0, w1)

        pltpu.sync_copy(pos_hbm, pos_v)
        pltpu.sync_copy(xt_hbm.at[:, pl.ds(b0, BBLK)], idx_v)

        lane = lax.iota(jnp.int32, L)
        rot = [(lane + k) & 7 for k in range(8)]
        dvec = [c * L + lane for c in range(DG)]
        dtv = [c * 2 + (lane >> 3) for c in range(DG)]
        wbase = [(lane & 7) * BBLK + rot[k] for k in range(8)]

        for j in range(2):
            pltpu.async_copy(tab_hbm.at[idx_v.at[j]], dsts[j], gsems[j])

        def t_pair(i, carry):
            for j in range(2):
                t = 2 * i + j
                dst, out_v = dsts[j], outs[j]
                pltpu.make_async_copy(
                    tab_hbm.at[idx_v.at[t]], dst, gsems[j]
                ).wait()
                @pl.when(i > 0)
                def _():
                    pltpu.make_async_copy(
                        out_v.at[:, pl.ds(0, 8 * BBLK)],
                        out_hbm.at[t, :, wid], wsems[j]
                    ).wait()

                pcs = [pos_v[t, pl.ds(c * L, L)] for c in range(DG)]

                @plsc.parallel_loop(0, BBLK, 8, unroll=4)
                def r_step(r0):
                    rsp = jnp.full((L,), r0, jnp.int32)
                    for k in range(8):
                        rvec = rsp + rot[k]
                        wv = rsp + wbase[k]
                        for c in range(DG):
                            vals = plsc.load_gather(dst, [rvec, dvec[c]]) + pcs[c]
                            plsc.store_scatter(out_v, [dtv[c], wv], vals)
                pltpu.async_copy(out_v.at[:, pl.ds(0, 8 * BBLK)],
                                 out_hbm.at[t, :, wid], wsems[j])

                @pl.when(t + 2 < MAXLEN)
                def _():
                    pltpu.async_copy(
                        tab_hbm.at[idx_v.at[t + 2]], dst, gsems[j]
                    )
            return carry

        lax.fori_loop(0, MAXLEN // 2, t_pair, 0)
        for j in range(2):
            pltpu.make_async_copy(
                outs[j].at[:, pl.ds(0, 8 * BBLK)],
                out_hbm.at[MAXLEN - 2 + j, :, wid], wsems[j]
            ).wait()

    return tok_pos_kernel


def kernel(x, token_table, pos_table):
    b, t = x.shape
    vocab, d = token_table.shape
    xt = x.T.astype(jnp.int32)
    tabp = jnp.pad(token_table, ((0, 0), (0, d)))
    o4 = _make_kernel(vocab, b)(xt, tabp, pos_table)
    o5 = o4.reshape(t, d // 8, NW, 8, BBLK)
    return o5.transpose(2, 4, 0, 1, 3).reshape(b, t, d)


# speedup vs baseline: 1.0715x; 1.0715x over previous
import functools

import jax
import jax.numpy as jnp
from jax import lax
from jax.experimental import pallas as pl
from jax.experimental.pallas import tpu as pltpu
from jax.experimental.pallas import tpu_sc as plsc

MAXLEN = 200
D = 64
L = 16
DG = D // L
BBLK = 128
RUN = 4

_info = plsc.get_sparse_core_info()
NC = _info.num_cores
NS = _info.num_subcores
NW = NC * NS


def _make_kernel(vocab, batch):
    assert batch == NW * BBLK

    mesh = plsc.VectorSubcoreMesh(core_axis_name="c", subcore_axis_name="s")

    @functools.partial(
        pl.kernel,
        mesh=mesh,
        compiler_params=pltpu.CompilerParams(
            use_tc_tiling_on_sc=False, needs_layout_passes=False
        ),
        out_type=jax.ShapeDtypeStruct((MAXLEN, D // 8, NW, 8 * BBLK), jnp.float32),
        scratch_types=[
            pltpu.VMEM((MAXLEN, D), jnp.float32),
            pltpu.VMEM((MAXLEN, BBLK), jnp.int32),
            pltpu.VMEM((BBLK, 2 * D), jnp.float32),
            pltpu.VMEM((BBLK, 2 * D), jnp.float32),
            pltpu.VMEM((D // 8, 8 * BBLK + 8), jnp.float32),
            pltpu.VMEM((D // 8, 8 * BBLK + 8), jnp.float32),
            pltpu.SemaphoreType.DMA,
            pltpu.SemaphoreType.DMA,
            pltpu.SemaphoreType.DMA,
            pltpu.SemaphoreType.DMA,
        ],
    )
    def tok_pos_kernel(xt_hbm, tab_hbm, pos_hbm, out_hbm,
                       pos_v, idx_v, dst0, dst1, out0, out1,
                       g0, g1, w0, w1):
        wid = lax.axis_index("s") * NC + lax.axis_index("c")
        b0 = wid * BBLK
        dsts = (dst0, dst1)
        outs = (out0, out1)
        gsems = (g0, g1)
        wsems = (w0, w1)

        pltpu.sync_copy(pos_hbm, pos_v)
        pltpu.sync_copy(xt_hbm.at[:, pl.ds(b0, BBLK)], idx_v)

        lane = lax.iota(jnp.int32, L)
        rot = [(lane + k) & 7 for k in range(8)]
        dvec = [c * L + lane for c in range(DG)]
        dtv = [c * 2 + (lane >> 3) for c in range(DG)]
        wbase = [(lane & 7) * BBLK + rot[k] for k in range(8)]

        for j in range(2):
            pltpu.async_copy(tab_hbm.at[idx_v.at[j]], dsts[j], gsems[j])

        def t_pair(i, carry):
            for j in range(2):
                t = 2 * i + j
                dst, out_v = dsts[j], outs[j]
                pltpu.make_async_copy(
                    tab_hbm.at[idx_v.at[t]], dst, gsems[j]
                ).wait()
                @pl.when(i > 0)
                def _():
                    pltpu.make_async_copy(
                        out_v.at[:, pl.ds(0, 8 * BBLK)],
                        out_hbm.at[t, :, wid], wsems[j]
                    ).wait()

                pcs = [pos_v[t, pl.ds(c * L, L)] for c in range(DG)]

                @plsc.parallel_loop(0, BBLK, 8, unroll=2)
                def r_step(r0):
                    rsp = jnp.full((L,), r0, jnp.int32)
                    for k in range(8):
                        rvec = rsp + rot[k]
                        wv = rsp + wbase[k]
                        for c in range(DG):
                            vals = plsc.load_gather(dst, [rvec, dvec[c]]) + pcs[c]
                            plsc.store_scatter(out_v, [dtv[c], wv], vals)
                pltpu.async_copy(out_v.at[:, pl.ds(0, 8 * BBLK)],
                                 out_hbm.at[t, :, wid], wsems[j])

                @pl.when(t + 2 < MAXLEN)
                def _():
                    pltpu.async_copy(
                        tab_hbm.at[idx_v.at[t + 2]], dst, gsems[j]
                    )
            return carry

        lax.fori_loop(0, MAXLEN // 2, t_pair, 0)
        for j in range(2):
            pltpu.make_async_copy(
                outs[j].at[:, pl.ds(0, 8 * BBLK)],
                out_hbm.at[MAXLEN - 2 + j, :, wid], wsems[j]
            ).wait()

    return tok_pos_kernel


def kernel(x, token_table, pos_table):
    b, t = x.shape
    vocab, d = token_table.shape
    xt = x.T.astype(jnp.int32)
    tabp = jnp.pad(token_table, ((0, 0), (0, d)))
    o4 = _make_kernel(vocab, b)(xt, tabp, pos_table)
    o5 = o4.reshape(t, d // 8, NW, 8, BBLK)
    return o5.transpose(2, 4, 0, 1, 3).reshape(b, t, d)
